# Initial kernel scaffold; baseline (speedup 1.0000x reference)
import functools

import jax
import jax.numpy as jnp
from jax import lax
from jax.experimental import pallas as pl
from jax.experimental.pallas import tpu as pltpu
from jax.experimental.pallas import tpu_sc as plsc

NN = 10000
EE = 320000
D = 128
NC = 2
NS = 16
NW = NC * NS
EW = EE // NW
NPAD = 10240
RPT = NPAD // NS
KA, CA = 100, 100
KD, CD = 125, 80
BLK = 1000

_mesh = plsc.VectorSubcoreMesh(core_axis_name="c", subcore_axis_name="s")


@functools.partial(
    pl.kernel,
    out_type=jax.ShapeDtypeStruct((NC, NPAD), jnp.float32),
    mesh=_mesh,
    scratch_types=[
        pltpu.VMEM((KD, CD), jnp.int32),
        pltpu.VMEM((CD,), jnp.float32),
        pltpu.VMEM((CD,), jnp.int32),
        pltpu.VMEM((RPT,), jnp.float32),
        pltpu.VMEM_SHARED((NPAD,), jnp.float32),
        pltpu.SemaphoreType.DMA,
    ],
)
def _sc_deg(dst_hbm, out_hbm, dst_v, ones_v, dummy_v, zb_v, acc, sem):
    cid = lax.axis_index("c")
    sid = lax.axis_index("s")
    wid = sid * NC + cid

    def fill(i, carry):
        ones_v[pl.ds(i * 16, 16)] = jnp.ones((16,), jnp.float32)
        return carry

    lax.fori_loop(0, CD // 16, fill, 0)

    def zfill(i, carry):
        zb_v[pl.ds(i * 16, 16)] = jnp.zeros((16,), jnp.float32)
        return carry

    lax.fori_loop(0, RPT // 16, zfill, 0)

    pltpu.sync_copy(dst_hbm.at[wid], dst_v)
    pltpu.sync_copy(zb_v, acc.at[pl.ds(sid * RPT, RPT)])
    plsc.subcore_barrier()

    def fire(j, carry):
        pltpu.async_copy(ones_v, acc.at[dst_v.at[j]], sem, add=True)
        return carry

    lax.fori_loop(0, KD, fire, 0)

    def drain(j, carry):
        pltpu.make_async_copy(dst_hbm.at[wid].at[pl.ds(0, CD)], dummy_v, sem).wait()
        return carry

    lax.fori_loop(0, KD, drain, 0)
    plsc.subcore_barrier()
    pltpu.sync_copy(
        acc.at[pl.ds(sid * RPT, RPT)], out_hbm.at[cid].at[pl.ds(sid * RPT, RPT)]
    )


@functools.partial(
    pl.kernel,
    out_type=jax.ShapeDtypeStruct((NC, NPAD, D), jnp.float32),
    mesh=_mesh,
    scratch_types=[
        pltpu.VMEM((KA, CA), jnp.int32),
        pltpu.VMEM((KA, CA), jnp.int32),
        pltpu.VMEM((CA, D), jnp.float32),
        pltpu.VMEM((CA, D), jnp.float32),
        pltpu.VMEM((80, D), jnp.float32),
        pltpu.VMEM_SHARED((NPAD, D), jnp.float32),
        pltpu.SemaphoreType.DMA,
        pltpu.SemaphoreType.DMA,
    ],
)
def _sc_agg(hs_hbm, src_hbm, dst_hbm, out_hbm, src_v, dst_v, gb0, gb1, zb, acc, sem0, sem1):
    cid = lax.axis_index("c")
    sid = lax.axis_index("s")
    wid = sid * NC + cid

    def zfill(i, carry):
        row = zb.at[i]
        for l in range(D // 16):
            row[pl.ds(l * 16, 16)] = jnp.zeros((16,), jnp.float32)
        return carry

    lax.fori_loop(0, 80, zfill, 0)

    pltpu.sync_copy(src_hbm.at[wid], src_v)
    pltpu.sync_copy(dst_hbm.at[wid], dst_v)
    base = sid * RPT
    for k in range(RPT // 80):
        pltpu.sync_copy(zb, acc.at[pl.ds(base + k * 80, 80)])
    plsc.subcore_barrier()

    pltpu.async_copy(hs_hbm.at[src_v.at[0]], gb0, sem0)
    pltpu.async_copy(hs_hbm.at[src_v.at[1]], gb1, sem1)

    def body(g, carry):
        j0 = 2 * g
        pltpu.make_async_copy(hs_hbm.at[src_v.at[0]], gb0, sem0).wait()
        pltpu.sync_copy(gb0, acc.at[dst_v.at[j0]], add=True)

        @pl.when(j0 + 2 < KA)
        def _():
            pltpu.async_copy(hs_hbm.at[src_v.at[j0 + 2]], gb0, sem0)

        pltpu.make_async_copy(hs_hbm.at[src_v.at[0]], gb1, sem1).wait()
        pltpu.sync_copy(gb1, acc.at[dst_v.at[j0 + 1]], add=True)

        @pl.when(j0 + 3 < KA)
        def _():
            pltpu.async_copy(hs_hbm.at[src_v.at[j0 + 3]], gb1, sem1)

        return carry

    lax.fori_loop(0, KA // 2, body, 0)
    plsc.subcore_barrier()
    pltpu.sync_copy(acc.at[pl.ds(base, RPT)], out_hbm.at[cid].at[pl.ds(base, RPT)])


def _deg_inv_sqrt(degp_ref):
    deg = degp_ref[0, :, 0] + degp_ref[1, :, 0] + 1.0
    return lax.rsqrt(deg)[:, None]


def _tc1_body(x_ref, w_ref, degp_ref, hs_ref):
    d = _deg_inv_sqrt(degp_ref)
    h = jnp.dot(x_ref[...], w_ref[...], preferred_element_type=jnp.float32)
    hs_ref[...] = h * d


def _tc2_body(acc_ref, hs_ref, degp_ref, w_ref, b_ref, a_ref, out_ref):
    d = _deg_inv_sqrt(degp_ref)
    pre = (acc_ref[0] + acc_ref[1] + hs_ref[...]) * d + b_ref[...]
    a = a_ref[0, 0]
    h1 = jnp.where(pre >= 0.0, pre, a * pre)
    out_ref[...] = jnp.dot(h1, w_ref[...], preferred_element_type=jnp.float32) * d


def _tc3_body(acc_ref, hs_ref, degp_ref, b_ref, a_ref, out_ref):
    d = _deg_inv_sqrt(degp_ref)
    pre = (acc_ref[0] + acc_ref[1] + hs_ref[...]) * d + b_ref[...]
    a = a_ref[0, 0]
    h2 = jnp.where(pre >= 0.0, pre, a * pre)
    m = jnp.max(h2, axis=1, keepdims=True)
    lse = jnp.log(jnp.sum(jnp.exp(h2 - m), axis=1, keepdims=True)) + m
    out_ref[...] = h2 - lse


_GRID = (NN // BLK,)
_row = pl.BlockSpec((BLK, D), lambda j: (j, 0))
_wspec = pl.BlockSpec((D, D), lambda j: (0, 0))
_degspec = pl.BlockSpec((NC, BLK, 1), lambda j: (0, j, 0))
_accspec = pl.BlockSpec((NC, BLK, D), lambda j: (0, j, 0))
_bspec = pl.BlockSpec((1, D), lambda j: (0, 0))
_aspec = pl.BlockSpec((1, 1), lambda j: (0, 0))
_rowout = jax.ShapeDtypeStruct((NN, D), jnp.float32)

_tc1 = pl.pallas_call(
    _tc1_body, grid=_GRID,
    in_specs=[_row, _wspec, _degspec],
    out_specs=_row, out_shape=_rowout,
)
_tc2 = pl.pallas_call(
    _tc2_body, grid=_GRID,
    in_specs=[_accspec, _row, _degspec, _wspec, _bspec, _aspec],
    out_specs=_row, out_shape=_rowout,
)
_tc3 = pl.pallas_call(
    _tc3_body, grid=_GRID,
    in_specs=[_accspec, _row, _degspec, _bspec, _aspec],
    out_specs=_row, out_shape=_rowout,
)


def kernel(x, edge_index, W1, b1, W2, b2, prelu_a):
    src = edge_index[0]
    dst = edge_index[1]
    src_a = src.reshape(NW, KA, CA)
    dst_a = dst.reshape(NW, KA, CA)
    dst_d = dst.reshape(NW, KD, CD)

    degp = _sc_deg(dst_d)
    degp3 = degp.reshape(NC, NPAD, 1)

    hs1 = _tc1(x, W1, degp3)
    acc1 = _sc_agg(hs1, src_a, dst_a)
    hs2 = _tc2(acc1, hs1, degp3, W2, b1.reshape(1, D), prelu_a.reshape(1, 1))
    acc2 = _sc_agg(hs2, src_a, dst_a)
    return _tc3(acc2, hs2, degp3, b2.reshape(1, D), prelu_a.reshape(1, 1))


# speedup vs baseline: 15.1045x; 15.1045x over previous
import functools

import jax
import jax.numpy as jnp
from jax import lax
from jax.experimental import pallas as pl
from jax.experimental.pallas import tpu as pltpu
from jax.experimental.pallas import tpu_sc as plsc

NN = 10000
EE = 320000
D = 128
NC = 2
NS = 16
NW = NC * NS
EW = EE // NW
NPAD = 10240
RPT = NPAD // NS
KA, CA = 100, 100
KD, CD = 125, 80
BLK = 1000

_mesh = plsc.VectorSubcoreMesh(core_axis_name="c", subcore_axis_name="s")


DW = 16


@functools.partial(
    pl.kernel,
    out_type=jax.ShapeDtypeStruct((NC, NPAD, DW), jnp.float32),
    mesh=_mesh,
    scratch_types=[
        pltpu.VMEM((KD, CD), jnp.int32),
        pltpu.VMEM((CD, DW), jnp.float32),
        pltpu.VMEM((RPT, DW), jnp.float32),
        pltpu.VMEM_SHARED((NPAD, DW), jnp.float32),
    ],
    compiler_params=pltpu.CompilerParams(use_tc_tiling_on_sc=False),
)
def _sc_deg(dst_hbm, out_hbm, dst_v, ones_v, zb_v, acc):
    cid = lax.axis_index("c")
    sid = lax.axis_index("s")
    wid = sid * NC + cid

    def fill(i, carry):
        ones_v.at[i][pl.ds(0, DW)] = jnp.ones((DW,), jnp.float32)
        return carry

    lax.fori_loop(0, CD, fill, 0)

    def zfill(i, carry):
        zb_v.at[i][pl.ds(0, DW)] = jnp.zeros((DW,), jnp.float32)
        return carry

    lax.fori_loop(0, RPT, zfill, 0)

    pltpu.sync_copy(dst_hbm.at[wid], dst_v)
    pltpu.sync_copy(zb_v, acc.at[pl.ds(sid * RPT, RPT)])
    plsc.subcore_barrier()

    def body(j, carry):
        pltpu.sync_copy(ones_v, acc.at[dst_v.at[j]], add=True)
        return carry

    lax.fori_loop(0, KD, body, 0)
    plsc.subcore_barrier()
    pltpu.sync_copy(
        acc.at[pl.ds(sid * RPT, RPT)], out_hbm.at[cid].at[pl.ds(sid * RPT, RPT)]
    )


DH = D // 2


@functools.partial(
    pl.kernel,
    out_type=jax.ShapeDtypeStruct((2, NC, NPAD, DH), jnp.float32),
    mesh=_mesh,
    scratch_types=[
        pltpu.VMEM((KA, CA), jnp.int32),
        pltpu.VMEM((KA, CA), jnp.int32),
        pltpu.VMEM((CA, DH), jnp.float32),
        pltpu.VMEM((80, DH), jnp.float32),
        pltpu.VMEM_SHARED((NPAD, DH), jnp.float32),
    ],
    compiler_params=pltpu.CompilerParams(use_tc_tiling_on_sc=False),
)
def _sc_agg(hsa_hbm, hsb_hbm, src_hbm, dst_hbm, out_hbm,
            src_v, dst_v, gb0, zb, acc):
    cid = lax.axis_index("c")
    sid = lax.axis_index("s")
    wid = sid * NC + cid
    base = sid * RPT

    def zfill(i, carry):
        row = zb.at[i]
        for l in range(DH // 16):
            row[pl.ds(l * 16, 16)] = jnp.zeros((16,), jnp.float32)
        return carry

    lax.fori_loop(0, 80, zfill, 0)

    pltpu.sync_copy(src_hbm.at[wid], src_v)
    pltpu.sync_copy(dst_hbm.at[wid], dst_v)

    for h, hs_hbm in enumerate((hsa_hbm, hsb_hbm)):
        for k in range(RPT // 80):
            pltpu.sync_copy(zb, acc.at[pl.ds(base + k * 80, 80)])
        plsc.subcore_barrier()

        def body(j, carry):
            pltpu.sync_copy(hs_hbm.at[src_v.at[j]], gb0)
            pltpu.sync_copy(gb0, acc.at[dst_v.at[j]], add=True)
            return carry

        lax.fori_loop(0, KA, body, 0)
        plsc.subcore_barrier()
        pltpu.sync_copy(
            acc.at[pl.ds(base, RPT)],
            out_hbm.at[h].at[cid].at[pl.ds(base, RPT)],
        )
        if h == 0:
            plsc.subcore_barrier()


def _deg_inv_sqrt(degp_ref):
    deg = degp_ref[0, :, 0] + degp_ref[1, :, 0] + 1.0
    return lax.rsqrt(deg)[:, None]


def _tc1_body(x_ref, w_ref, degp_ref, hs_ref):
    d = _deg_inv_sqrt(degp_ref)
    h = jnp.dot(x_ref[...], w_ref[...], preferred_element_type=jnp.float32)
    hs_ref[...] = h * d


def _agg_full(acc_ref):
    return jnp.concatenate(
        [acc_ref[0, 0] + acc_ref[0, 1], acc_ref[1, 0] + acc_ref[1, 1]], axis=1
    )


def _tc2_body(acc_ref, hs_ref, degp_ref, w_ref, b_ref, a_ref, out_ref):
    d = _deg_inv_sqrt(degp_ref)
    pre = (_agg_full(acc_ref) + hs_ref[...]) * d + b_ref[...]
    a = a_ref[0, 0]
    h1 = jnp.where(pre >= 0.0, pre, a * pre)
    out_ref[...] = jnp.dot(h1, w_ref[...], preferred_element_type=jnp.float32) * d


def _tc3_body(acc_ref, hs_ref, degp_ref, b_ref, a_ref, out_ref):
    d = _deg_inv_sqrt(degp_ref)
    pre = (_agg_full(acc_ref) + hs_ref[...]) * d + b_ref[...]
    a = a_ref[0, 0]
    h2 = jnp.where(pre >= 0.0, pre, a * pre)
    m = jnp.max(h2, axis=1, keepdims=True)
    lse = jnp.log(jnp.sum(jnp.exp(h2 - m), axis=1, keepdims=True)) + m
    out_ref[...] = h2 - lse


_GRID = (NN // BLK,)
_row = pl.BlockSpec((BLK, D), lambda j: (j, 0))
_wspec = pl.BlockSpec((D, D), lambda j: (0, 0))
_degspec = pl.BlockSpec((NC, BLK, DW), lambda j: (0, j, 0))
_accspec = pl.BlockSpec((2, NC, BLK, DH), lambda j: (0, 0, j, 0))
_bspec = pl.BlockSpec((1, D), lambda j: (0, 0))
_aspec = pl.BlockSpec((1, 1), lambda j: (0, 0))
_rowout = jax.ShapeDtypeStruct((NN, D), jnp.float32)

_tc1 = pl.pallas_call(
    _tc1_body, grid=_GRID,
    in_specs=[_row, _wspec, _degspec],
    out_specs=_row, out_shape=_rowout,
)
_tc2 = pl.pallas_call(
    _tc2_body, grid=_GRID,
    in_specs=[_accspec, _row, _degspec, _wspec, _bspec, _aspec],
    out_specs=_row, out_shape=_rowout,
)
_tc3 = pl.pallas_call(
    _tc3_body, grid=_GRID,
    in_specs=[_accspec, _row, _degspec, _bspec, _aspec],
    out_specs=_row, out_shape=_rowout,
)


def kernel(x, edge_index, W1, b1, W2, b2, prelu_a):
    src = edge_index[0]
    dst = edge_index[1]
    src_a = src.reshape(NW, KA, CA)
    dst_a = dst.reshape(NW, KA, CA)
    dst_d = dst.reshape(NW, KD, CD)

    degp = _sc_deg(dst_d)

    hs1 = _tc1(x, W1, degp)
    acc1 = _sc_agg(hs1[:, :DH], hs1[:, DH:], src_a, dst_a)
    hs2 = _tc2(acc1, hs1, degp, W2, b1.reshape(1, D), prelu_a.reshape(1, 1))
    acc2 = _sc_agg(hs2[:, :DH], hs2[:, DH:], src_a, dst_a)
    return _tc3(acc2, hs2, degp, b2.reshape(1, D), prelu_a.reshape(1, 1))


# speedup vs baseline: 28.2031x; 1.8672x over previous
import functools

import jax
import jax.numpy as jnp
from jax import lax
from jax.experimental import pallas as pl
from jax.experimental.pallas import tpu as pltpu
from jax.experimental.pallas import tpu_sc as plsc

NN = 10000
EE = 320000
D = 128
NC = 2
NS = 16
NW = NC * NS
EW = EE // NW
NPAD = 10240
RPT = NPAD // NS
KA, CA = 100, 100
KD, CD = 125, 80
BLK = 1000

_mesh = plsc.VectorSubcoreMesh(core_axis_name="c", subcore_axis_name="s")


DW = 16


@functools.partial(
    pl.kernel,
    out_type=jax.ShapeDtypeStruct((NC, NPAD, DW), jnp.float32),
    mesh=_mesh,
    scratch_types=[
        pltpu.VMEM((KD, CD), jnp.int32),
        pltpu.VMEM((CD, DW), jnp.float32),
        pltpu.VMEM((RPT, DW), jnp.float32),
        pltpu.VMEM_SHARED((NPAD, DW), jnp.float32),
    ],
    compiler_params=pltpu.CompilerParams(use_tc_tiling_on_sc=False),
)
def _sc_deg(dst_hbm, out_hbm, dst_v, ones_v, zb_v, acc):
    cid = lax.axis_index("c")
    sid = lax.axis_index("s")
    wid = sid * NC + cid

    def fill(i, carry):
        ones_v.at[i][pl.ds(0, DW)] = jnp.ones((DW,), jnp.float32)
        return carry

    lax.fori_loop(0, CD, fill, 0)

    def zfill(i, carry):
        zb_v.at[i][pl.ds(0, DW)] = jnp.zeros((DW,), jnp.float32)
        return carry

    lax.fori_loop(0, RPT, zfill, 0)

    pltpu.sync_copy(dst_hbm.at[wid], dst_v)
    pltpu.sync_copy(zb_v, acc.at[pl.ds(sid * RPT, RPT)])
    plsc.subcore_barrier()

    def body(j, carry):
        pltpu.sync_copy(ones_v, acc.at[dst_v.at[j]], add=True)
        return carry

    lax.fori_loop(0, KD, body, 0)
    plsc.subcore_barrier()
    pltpu.sync_copy(
        acc.at[pl.ds(sid * RPT, RPT)], out_hbm.at[cid].at[pl.ds(sid * RPT, RPT)]
    )


DH = D // 2
NB = 4


@functools.partial(
    pl.kernel,
    out_type=jax.ShapeDtypeStruct((2, NC, NPAD, DH), jnp.float32),
    mesh=_mesh,
    scratch_types=[
        pltpu.VMEM((KA, CA), jnp.int32),
        pltpu.VMEM((KA, CA), jnp.int32),
        pltpu.VMEM((NB, CA, DH), jnp.float32),
        pltpu.VMEM((80, DH), jnp.float32),
        pltpu.VMEM_SHARED((NPAD, DH), jnp.float32),
        [pltpu.SemaphoreType.DMA] * NB,
        [pltpu.SemaphoreType.DMA] * NB,
    ],
    compiler_params=pltpu.CompilerParams(use_tc_tiling_on_sc=False),
)
def _sc_agg(hsa_hbm, hsb_hbm, src_hbm, dst_hbm, out_hbm,
            src_v, dst_v, gb, zb, acc, gsem, ssem):
    cid = lax.axis_index("c")
    sid = lax.axis_index("s")
    wid = sid * NC + cid
    base = sid * RPT

    def zfill(i, carry):
        row = zb.at[i]
        for l in range(DH // 16):
            row[pl.ds(l * 16, 16)] = jnp.zeros((16,), jnp.float32)
        return carry

    lax.fori_loop(0, 80, zfill, 0)

    pltpu.sync_copy(src_hbm.at[wid], src_v)
    pltpu.sync_copy(dst_hbm.at[wid], dst_v)

    for h, hs_hbm in enumerate((hsa_hbm, hsb_hbm)):
        for k in range(RPT // 80):
            pltpu.sync_copy(zb, acc.at[pl.ds(base + k * 80, 80)])
        plsc.subcore_barrier()

        for b in range(NB):
            pltpu.async_copy(hs_hbm.at[src_v.at[b]], gb.at[b], gsem[b])

        def body(g, carry):
            for b in range(NB):
                j = NB * g + b
                pltpu.make_async_copy(hs_hbm.at[src_v.at[0]], gb.at[b], gsem[b]).wait()
                pltpu.async_copy(gb.at[b], acc.at[dst_v.at[j]], ssem[b], add=True)

                @pl.when(j + NB < KA)
                def _():
                    pltpu.make_async_copy(gb.at[b], acc.at[dst_v.at[0]], ssem[b]).wait()
                    pltpu.async_copy(hs_hbm.at[src_v.at[j + NB]], gb.at[b], gsem[b])

            return carry

        lax.fori_loop(0, KA // NB, body, 0)
        for b in range(NB):
            pltpu.make_async_copy(gb.at[b], acc.at[dst_v.at[0]], ssem[b]).wait()
        plsc.subcore_barrier()
        pltpu.sync_copy(
            acc.at[pl.ds(base, RPT)],
            out_hbm.at[h].at[cid].at[pl.ds(base, RPT)],
        )
        if h == 0:
            plsc.subcore_barrier()


def _deg_inv_sqrt(degp_ref):
    deg = degp_ref[0, :, 0] + degp_ref[1, :, 0] + 1.0
    return lax.rsqrt(deg)[:, None]


def _tc1_body(x_ref, w_ref, degp_ref, hs_ref):
    d = _deg_inv_sqrt(degp_ref)
    h = jnp.dot(x_ref[...], w_ref[...], preferred_element_type=jnp.float32)
    hs_ref[...] = h * d


def _agg_full(acc_ref):
    return jnp.concatenate(
        [acc_ref[0, 0] + acc_ref[0, 1], acc_ref[1, 0] + acc_ref[1, 1]], axis=1
    )


def _tc2_body(acc_ref, hs_ref, degp_ref, w_ref, b_ref, a_ref, out_ref):
    d = _deg_inv_sqrt(degp_ref)
    pre = (_agg_full(acc_ref) + hs_ref[...]) * d + b_ref[...]
    a = a_ref[0, 0]
    h1 = jnp.where(pre >= 0.0, pre, a * pre)
    out_ref[...] = jnp.dot(h1, w_ref[...], preferred_element_type=jnp.float32) * d


def _tc3_body(acc_ref, hs_ref, degp_ref, b_ref, a_ref, out_ref):
    d = _deg_inv_sqrt(degp_ref)
    pre = (_agg_full(acc_ref) + hs_ref[...]) * d + b_ref[...]
    a = a_ref[0, 0]
    h2 = jnp.where(pre >= 0.0, pre, a * pre)
    m = jnp.max(h2, axis=1, keepdims=True)
    lse = jnp.log(jnp.sum(jnp.exp(h2 - m), axis=1, keepdims=True)) + m
    out_ref[...] = h2 - lse


_GRID = (NN // BLK,)
_row = pl.BlockSpec((BLK, D), lambda j: (j, 0))
_wspec = pl.BlockSpec((D, D), lambda j: (0, 0))
_degspec = pl.BlockSpec((NC, BLK, DW), lambda j: (0, j, 0))
_accspec = pl.BlockSpec((2, NC, BLK, DH), lambda j: (0, 0, j, 0))
_bspec = pl.BlockSpec((1, D), lambda j: (0, 0))
_aspec = pl.BlockSpec((1, 1), lambda j: (0, 0))
_rowout = jax.ShapeDtypeStruct((NN, D), jnp.float32)

_tc1 = pl.pallas_call(
    _tc1_body, grid=_GRID,
    in_specs=[_row, _wspec, _degspec],
    out_specs=_row, out_shape=_rowout,
)
_tc2 = pl.pallas_call(
    _tc2_body, grid=_GRID,
    in_specs=[_accspec, _row, _degspec, _wspec, _bspec, _aspec],
    out_specs=_row, out_shape=_rowout,
)
_tc3 = pl.pallas_call(
    _tc3_body, grid=_GRID,
    in_specs=[_accspec, _row, _degspec, _bspec, _aspec],
    out_specs=_row, out_shape=_rowout,
)


def kernel(x, edge_index, W1, b1, W2, b2, prelu_a):
    src = edge_index[0]
    dst = edge_index[1]
    src_a = src.reshape(NW, KA, CA)
    dst_a = dst.reshape(NW, KA, CA)
    dst_d = dst.reshape(NW, KD, CD)

    degp = _sc_deg(dst_d)

    hs1 = _tc1(x, W1, degp)
    acc1 = _sc_agg(hs1[:, :DH], hs1[:, DH:], src_a, dst_a)
    hs2 = _tc2(acc1, hs1, degp, W2, b1.reshape(1, D), prelu_a.reshape(1, 1))
    acc2 = _sc_agg(hs2[:, :DH], hs2[:, DH:], src_a, dst_a)
    return _tc3(acc2, hs2, degp, b2.reshape(1, D), prelu_a.reshape(1, 1))
